# baseline (speedup 1.0000x reference)
import functools

import jax
import jax.numpy as jnp
from jax import lax
from jax.experimental import pallas as pl
from jax.experimental.pallas import tpu as pltpu
from jax.experimental.pallas import tpu_sc as plsc

B = 16384
D = 16
NC = 2
NS = 16
NW = NC * NS
BPW = B // NW
CHUNK = 128
NCHUNK = BPW // CHUNK


def _gather_body(uidx_hbm, iidx_hbm, utab_hbm, itab_hbm, uout_hbm, iout_hbm,
                 uidx_v, iidx_v, urows_v, irows_v, usem, isem):
    wid = lax.axis_index("s") * NC + lax.axis_index("c")
    base = wid * BPW
    pltpu.sync_copy(uidx_hbm.at[pl.ds(wid * NCHUNK, NCHUNK)], uidx_v)
    pltpu.sync_copy(iidx_hbm.at[pl.ds(wid * NCHUNK, NCHUNK)], iidx_v)
    ucopies = []
    icopies = []
    for j in range(NCHUNK):
        ucopies.append(pltpu.async_copy(
            utab_hbm.at[uidx_v.at[j]], urows_v.at[pl.ds(j * CHUNK, CHUNK)],
            usem))
        icopies.append(pltpu.async_copy(
            itab_hbm.at[iidx_v.at[j]], irows_v.at[pl.ds(j * CHUNK, CHUNK)],
            isem))
    for c in ucopies:
        c.wait()
    pltpu.sync_copy(urows_v, uout_hbm.at[pl.ds(base, BPW)])
    for c in icopies:
        c.wait()
    pltpu.sync_copy(irows_v, iout_hbm.at[pl.ds(base, BPW)])


@functools.cache
def _sc_gather():
    return functools.partial(
        pl.kernel,
        out_type=(
            jax.ShapeDtypeStruct((B, D), jnp.float32),
            jax.ShapeDtypeStruct((B, D), jnp.float32),
        ),
        mesh=plsc.VectorSubcoreMesh(
            core_axis_name="c", subcore_axis_name="s", num_cores=NC,
            num_subcores=NS),
        scratch_types=[
            pltpu.VMEM((NCHUNK, CHUNK), jnp.int32),
            pltpu.VMEM((NCHUNK, CHUNK), jnp.int32),
            pltpu.VMEM((BPW, D), jnp.float32),
            pltpu.VMEM((BPW, D), jnp.float32),
            pltpu.SemaphoreType.DMA,
            pltpu.SemaphoreType.DMA,
        ],
        compiler_params=pltpu.CompilerParams(use_tc_tiling_on_sc=False),
    )(_gather_body)


BM = 2048


def _mlp_body(ue_ref, ie_ref, w1u_ref, w1i_ref, b1_ref, w2_ref, b2_ref,
              w3_ref, b3_ref, out_ref):
    h1 = jnp.dot(ue_ref[...], w1u_ref[...],
                 preferred_element_type=jnp.float32)
    h1 += jnp.dot(ie_ref[...], w1i_ref[...],
                  preferred_element_type=jnp.float32)
    h1 = jnp.maximum(h1 + b1_ref[...], 0.0)
    h2 = jnp.maximum(
        jnp.dot(h1, w2_ref[...], preferred_element_type=jnp.float32)
        + b2_ref[...], 0.0)
    out_ref[...] = (
        jnp.dot(h2, w3_ref[...], preferred_element_type=jnp.float32)
        + b3_ref[...])


def _full(shape):
    return pl.BlockSpec(shape, lambda i: (0,) * len(shape))


_mlp = pl.pallas_call(
    _mlp_body,
    grid=(B // BM,),
    in_specs=[
        pl.BlockSpec((BM, D), lambda i: (i, 0)),
        pl.BlockSpec((BM, D), lambda i: (i, 0)),
        _full((D, 64)),
        _full((D, 64)),
        _full((1, 64)),
        _full((64, 32)),
        _full((1, 32)),
        _full((32, 1)),
        _full((1, 1)),
    ],
    out_specs=pl.BlockSpec((BM, 1), lambda i: (i, 0)),
    out_shape=jax.ShapeDtypeStruct((B, 1), jnp.float32),
)


def kernel(user, item, user_table, item_table, W1, b1, W2, b2, W3, b3):
    uidx = user.astype(jnp.int32).reshape(B // CHUNK, CHUNK)
    iidx = item.astype(jnp.int32).reshape(B // CHUNK, CHUNK)
    ue, ie = _sc_gather()(uidx, iidx, user_table, item_table)
    w1u = W1[:, :D].T
    w1i = W1[:, D:].T
    return _mlp(ue, ie, w1u, w1i, b1.reshape(1, 64), W2.T, b2.reshape(1, 32),
                W3.T, b3.reshape(1, 1))


# speedup vs baseline: 6.1718x; 6.1718x over previous
import functools

import jax
import jax.numpy as jnp
from jax import lax
from jax.experimental import pallas as pl
from jax.experimental.pallas import tpu as pltpu
from jax.experimental.pallas import tpu_sc as plsc

B = 16384
D = 16
NC = 2
NS = 16
NW = NC * NS
BPW = B // NW
K = 8
NG = BPW // K
SLAB = 128


def _gather_body(uidx_hbm, iidx_hbm, utab_hbm, itab_hbm, uout_hbm, iout_hbm,
                 uidx_v, iidx_v, urows_v, irows_v,
                 uslab, islab, usemA, usemB, isemA, isemB):
    wid = lax.axis_index("s") * NC + lax.axis_index("c")
    base = wid * BPW
    pltpu.sync_copy(uidx_hbm.at[pl.ds(base, BPW)], uidx_v)
    pltpu.sync_copy(iidx_hbm.at[pl.ds(base, BPW)], iidx_v)
    uidx_s = uidx_v
    iidx_s = iidx_v

    rows16 = lax.iota(jnp.int32, 16)

    def fire8(tab, vec, lane0, slab, sem, buf):
        for j in range(K):
            v = vec[lane0 + j]
            off = pl.multiple_of((v // SLAB) * SLAB, SLAB)
            pltpu.async_copy(tab.at[:, pl.ds(off, SLAB)],
                             slab.at[buf, j], sem)

    def drain8(tab, vec, lane0, slab, sem, rows, ibase, buf):
        for j in range(K):
            pltpu.make_async_copy(tab.at[:, pl.ds(0, SLAB)],
                                  slab.at[buf, j], sem).wait()
            v = vec[lane0 + j]
            lane = jnp.full((16,), v % SLAB, jnp.int32)
            emb = plsc.load_gather(slab.at[buf, j], [rows16, lane])
            col = jnp.full((16,), ibase + j, jnp.int32)
            plsc.store_scatter(rows, [rows16, col], emb)

    def fire(t, lane0, buf):
        uvec = uidx_v[pl.ds(t * 16, 16)]
        ivec = iidx_v[pl.ds(t * 16, 16)]
        fire8(utab_hbm, uvec, lane0, uslab, usemA if buf == 0 else usemB,
              buf)
        fire8(itab_hbm, ivec, lane0, islab, isemA if buf == 0 else isemB,
              buf)

    def drain(t, lane0, buf):
        uvec = uidx_v[pl.ds(t * 16, 16)]
        ivec = iidx_v[pl.ds(t * 16, 16)]
        ibase = t * 16 + lane0
        drain8(utab_hbm, uvec, lane0, uslab,
               usemA if buf == 0 else usemB, urows_v, ibase, buf)
        drain8(itab_hbm, ivec, lane0, islab,
               isemA if buf == 0 else isemB, irows_v, ibase, buf)

    NT = BPW // 16

    fire(0, 0, 0)

    def body(t, carry):
        fire(t, 8, 1)
        drain(t, 0, 0)

        @pl.when(t < NT - 1)
        def _():
            fire(t + 1, 0, 0)

        drain(t, 8, 1)
        return carry

    lax.fori_loop(0, NT, body, 0)

    pltpu.sync_copy(urows_v, uout_hbm.at[:, pl.ds(base, BPW)])
    pltpu.sync_copy(irows_v, iout_hbm.at[:, pl.ds(base, BPW)])


@functools.cache
def _sc_gather():
    return functools.partial(
        pl.kernel,
        out_type=(
            jax.ShapeDtypeStruct((D, B), jnp.float32),
            jax.ShapeDtypeStruct((D, B), jnp.float32),
        ),
        mesh=plsc.VectorSubcoreMesh(
            core_axis_name="c", subcore_axis_name="s", num_cores=NC,
            num_subcores=NS),
        scratch_types=[
            pltpu.VMEM((BPW,), jnp.int32),
            pltpu.VMEM((BPW,), jnp.int32),
            pltpu.VMEM((D, BPW), jnp.float32),
            pltpu.VMEM((D, BPW), jnp.float32),
            pltpu.VMEM((2, K, D, SLAB), jnp.float32),
            pltpu.VMEM((2, K, D, SLAB), jnp.float32),
            pltpu.SemaphoreType.DMA,
            pltpu.SemaphoreType.DMA,
            pltpu.SemaphoreType.DMA,
            pltpu.SemaphoreType.DMA,
        ],
        compiler_params=pltpu.CompilerParams(needs_layout_passes=False),
    )(_gather_body)


BM = 2048


def _mlp_body(ue_ref, ie_ref, w1u_ref, w1i_ref, b1_ref, w2_ref, b2_ref,
              w3_ref, b3_ref, out_ref):
    h1 = jnp.dot(w1u_ref[...], ue_ref[...],
                 preferred_element_type=jnp.float32)
    h1 += jnp.dot(w1i_ref[...], ie_ref[...],
                  preferred_element_type=jnp.float32)
    h1 = jnp.maximum(h1 + b1_ref[...], 0.0)
    h2 = jnp.maximum(
        jnp.dot(w2_ref[...], h1, preferred_element_type=jnp.float32)
        + b2_ref[...], 0.0)
    out_ref[...] = (
        jnp.dot(w3_ref[...], h2, preferred_element_type=jnp.float32)
        + b3_ref[...])


def _full(shape):
    return pl.BlockSpec(shape, lambda i: (0,) * len(shape))


@functools.cache
def _mlp():
    return pl.pallas_call(
        _mlp_body,
        grid=(B // BM,),
        in_specs=[
            pl.BlockSpec((D, BM), lambda i: (0, i)),
            pl.BlockSpec((D, BM), lambda i: (0, i)),
            _full((64, D)),
            _full((64, D)),
            _full((64, 1)),
            _full((32, 64)),
            _full((32, 1)),
            _full((1, 32)),
            _full((1, 1)),
        ],
        out_specs=pl.BlockSpec((1, BM), lambda i: (0, i)),
        out_shape=jax.ShapeDtypeStruct((1, B), jnp.float32),
    )


def kernel(user, item, user_table, item_table, W1, b1, W2, b2, W3, b3):
    uidx = user.astype(jnp.int32)
    iidx = item.astype(jnp.int32)
    ue, ie = _sc_gather()(uidx, iidx, user_table.T, item_table.T)
    outT = _mlp()(ue, ie, W1[:, :D], W1[:, D:], b1.reshape(64, 1), W2,
                  b2.reshape(32, 1), W3, b3.reshape(1, 1))
    return outT.reshape(B, 1)


# speedup vs baseline: 6.4853x; 1.0508x over previous
import functools

import jax
import jax.numpy as jnp
from jax import lax
from jax.experimental import pallas as pl
from jax.experimental.pallas import tpu as pltpu
from jax.experimental.pallas import tpu_sc as plsc

B = 16384
D = 16
NC = 2
NS = 16
NW = NC * NS
BPW = B // NW
K = 8
NG = BPW // K
SLAB = 128


def _gather_body(uidx_hbm, iidx_hbm, utab_hbm, itab_hbm, uout_hbm, iout_hbm,
                 uidx_v, iidx_v, urows_v, irows_v, uslab, islab,
                 usem0, usem1, usem2, isem0, isem1, isem2):
    wid = lax.axis_index("s") * NC + lax.axis_index("c")
    base = wid * BPW
    pltpu.sync_copy(uidx_hbm.at[pl.ds(base, BPW)], uidx_v)
    pltpu.sync_copy(iidx_hbm.at[pl.ds(base, BPW)], iidx_v)

    rows16 = lax.iota(jnp.int32, 16)
    usems = (usem0, usem1, usem2)
    isems = (isem0, isem1, isem2)


    def fire_grp(tv, half, buf):
        uvec = uidx_v[pl.ds(tv * 16, 16)]
        ivec = iidx_v[pl.ds(tv * 16, 16)]
        for j in range(K):
            v = uvec[half + j]
            off = pl.multiple_of((v // SLAB) * SLAB, SLAB)
            pltpu.async_copy(utab_hbm.at[:, pl.ds(off, SLAB)],
                             uslab.at[buf, j], usems[buf])
            w = ivec[half + j]
            offi = pl.multiple_of((w // SLAB) * SLAB, SLAB)
            pltpu.async_copy(itab_hbm.at[:, pl.ds(offi, SLAB)],
                             islab.at[buf, j], isems[buf])

    def drain_grp(tv, half, buf):
        uvec = uidx_v[pl.ds(tv * 16, 16)]
        ivec = iidx_v[pl.ds(tv * 16, 16)]
        ibase = tv * 16 + half
        for j in range(K):
            col = jnp.full((16,), ibase + j, jnp.int32)
            pltpu.make_async_copy(utab_hbm.at[:, pl.ds(0, SLAB)],
                                  uslab.at[buf, j], usems[buf]).wait()
            v = uvec[half + j]
            lane = jnp.full((16,), v % SLAB, jnp.int32)
            emb = plsc.load_gather(uslab.at[buf, j], [rows16, lane])
            plsc.store_scatter(urows_v, [rows16, col], emb)
            pltpu.make_async_copy(itab_hbm.at[:, pl.ds(0, SLAB)],
                                  islab.at[buf, j], isems[buf]).wait()
            w = ivec[half + j]
            lanei = jnp.full((16,), w % SLAB, jnp.int32)
            embi = plsc.load_gather(islab.at[buf, j], [rows16, lanei])
            plsc.store_scatter(irows_v, [rows16, col], embi)

    fire_grp(0, 0, 0)
    fire_grp(0, 8, 1)

    def body(s, carry):
        for k in range(6):
            fire_grp(3 * s + (k + 2) // 2, ((k + 2) % 2) * 8, (k + 2) % 3)
            drain_grp(3 * s + k // 2, (k % 2) * 8, k % 3)
        return carry

    lax.fori_loop(0, 10, body, 0)

    fire_grp(31, 0, 2)
    drain_grp(30, 0, 0)
    fire_grp(31, 8, 0)
    drain_grp(30, 8, 1)
    drain_grp(31, 0, 2)
    drain_grp(31, 8, 0)

    pltpu.sync_copy(urows_v, uout_hbm.at[:, pl.ds(base, BPW)])
    pltpu.sync_copy(irows_v, iout_hbm.at[:, pl.ds(base, BPW)])


@functools.cache
def _sc_gather():
    return functools.partial(
        pl.kernel,
        out_type=(
            jax.ShapeDtypeStruct((D, B), jnp.float32),
            jax.ShapeDtypeStruct((D, B), jnp.float32),
        ),
        mesh=plsc.VectorSubcoreMesh(
            core_axis_name="c", subcore_axis_name="s", num_cores=NC,
            num_subcores=NS),
        scratch_types=[
            pltpu.VMEM((BPW,), jnp.int32),
            pltpu.VMEM((BPW,), jnp.int32),
            pltpu.VMEM((D, BPW), jnp.float32),
            pltpu.VMEM((D, BPW), jnp.float32),
            pltpu.VMEM((3, K, D, SLAB), jnp.float32),
            pltpu.VMEM((3, K, D, SLAB), jnp.float32),
            pltpu.SemaphoreType.DMA,
            pltpu.SemaphoreType.DMA,
            pltpu.SemaphoreType.DMA,
            pltpu.SemaphoreType.DMA,
            pltpu.SemaphoreType.DMA,
            pltpu.SemaphoreType.DMA,
        ],
        compiler_params=pltpu.CompilerParams(needs_layout_passes=False),
    )(_gather_body)


BM = 2048


def _mlp_body(ue_ref, ie_ref, w1u_ref, w1i_ref, b1_ref, w2_ref, b2_ref,
              w3_ref, b3_ref, out_ref):
    h1 = jnp.dot(w1u_ref[...], ue_ref[...],
                 preferred_element_type=jnp.float32)
    h1 += jnp.dot(w1i_ref[...], ie_ref[...],
                  preferred_element_type=jnp.float32)
    h1 = jnp.maximum(h1 + b1_ref[...], 0.0)
    h2 = jnp.maximum(
        jnp.dot(w2_ref[...], h1, preferred_element_type=jnp.float32)
        + b2_ref[...], 0.0)
    out_ref[...] = (
        jnp.dot(w3_ref[...], h2, preferred_element_type=jnp.float32)
        + b3_ref[...])


def _full(shape):
    return pl.BlockSpec(shape, lambda i: (0,) * len(shape))


@functools.cache
def _mlp():
    return pl.pallas_call(
        _mlp_body,
        grid=(B // BM,),
        in_specs=[
            pl.BlockSpec((D, BM), lambda i: (0, i)),
            pl.BlockSpec((D, BM), lambda i: (0, i)),
            _full((64, D)),
            _full((64, D)),
            _full((64, 1)),
            _full((32, 64)),
            _full((32, 1)),
            _full((1, 32)),
            _full((1, 1)),
        ],
        out_specs=pl.BlockSpec((1, BM), lambda i: (0, i)),
        out_shape=jax.ShapeDtypeStruct((1, B), jnp.float32),
    )


def kernel(user, item, user_table, item_table, W1, b1, W2, b2, W3, b3):
    uidx = user.astype(jnp.int32)
    iidx = item.astype(jnp.int32)
    ue, ie = _sc_gather()(uidx, iidx, user_table.T, item_table.T)
    outT = _mlp()(ue, ie, W1[:, :D], W1[:, D:], b1.reshape(64, 1), W2,
                  b2.reshape(32, 1), W3, b3.reshape(1, 1))
    return outT.reshape(B, 1)
